# baseline (speedup 1.0000x reference)
import functools

import jax
import jax.numpy as jnp
from jax import lax
from jax.experimental import pallas as pl
from jax.experimental.pallas import tpu as pltpu
from jax.experimental.pallas import tpu_sc as plsc

B = 4096
L = 200
D = 64
NUM_WORKERS = 32
RPW = B // NUM_WORKERS
CA = 104
CB = L - CA
LANES = 16
NCHUNK = D // LANES


def _worker_id():
    return lax.axis_index("s") * 2 + lax.axis_index("c")


def _body(rowsa_hbm, rowsb_hbm, coff_hbm, len_hbm, table_hbm, out_hbm,
          idxa_v, idxb_v, coff_v, len_v, inv_v, bufa, bufb, outw, sems):
    wid = _worker_id()

    pltpu.sync_copy(rowsa_hbm.at[wid], idxa_v)
    pltpu.sync_copy(rowsb_hbm.at[wid], idxb_v)
    pltpu.sync_copy(coff_hbm.at[wid], coff_v)
    pltpu.sync_copy(len_hbm.at[wid], len_v)

    for g in range(RPW // LANES):
        lens16 = len_v[pl.ds(g * LANES, LANES)]
        inv_v[pl.ds(g * LANES, LANES)] = (
            1.0 / jnp.maximum(lens16, 1).astype(jnp.float32))

    lane = lax.broadcasted_iota(jnp.int32, (LANES,), 0)
    zero = lane * 0
    sem0, sem1 = sems

    def issue(r, slot, sem):
        pltpu.async_copy(table_hbm.at[idxa_v.at[r]], bufa.at[slot], sem)
        pltpu.async_copy(table_hbm.at[idxb_v.at[r]], bufb.at[slot], sem)

    def drain(slot, sem):
        pltpu.make_async_copy(table_hbm.at[pl.ds(0, CA)],
                              bufa.at[slot], sem).wait()
        pltpu.make_async_copy(table_hbm.at[pl.ds(0, CB)],
                              bufb.at[slot], sem).wait()

    def accumulate(r, slot):
        r_b = zero + r
        slot_b = zero + slot

        def make_acc(buf, base):
            def acc_body(t, accs):
                coff = plsc.load_gather(coff_v, [zero + base + t, r_b])
                new = []
                for c in range(NCHUNK):
                    col = coff + (c * LANES) + lane
                    new.append(accs[c] + plsc.load_gather(
                        buf, [slot_b, zero + t, col]))
                return tuple(new)
            return acc_body

        accs = tuple(jnp.zeros((LANES,), jnp.float32) for _ in range(NCHUNK))
        accs = lax.fori_loop(0, CA, make_acc(bufa, 0), accs, unroll=4)
        accs = lax.fori_loop(0, CB, make_acc(bufb, CA), accs, unroll=4)

        sinv = plsc.load_gather(inv_v, [r_b])
        for c in range(NCHUNK):
            outw[r, pl.ds(c * LANES, LANES)] = accs[c] * sinv

    issue(0, 0, sem0)

    def pair_body(p, carry):
        r0 = 2 * p
        r1 = r0 + 1
        issue(r1, 1, sem1)
        drain(0, sem0)
        accumulate(r0, 0)
        issue(jnp.minimum(r1 + 1, RPW - 1), 0, sem0)
        drain(1, sem1)
        accumulate(r1, 1)
        return carry

    lax.fori_loop(0, RPW // 2, pair_body, 0)
    drain(0, sem0)
    pltpu.sync_copy(outw, out_hbm.at[pl.ds(wid * RPW, RPW)])


@functools.partial(jax.jit, static_argnames=("interpret",))
def _run(tokens, token_lengths, table, interpret=False):
    mesh = plsc.VectorSubcoreMesh(core_axis_name="c", subcore_axis_name="s",
                                  num_cores=2, num_subcores=16)
    tab2 = table.reshape(500000, 2 * D)
    rows = tokens >> 1
    rowsa = rows[:, :CA].reshape(NUM_WORKERS, RPW, CA)
    rowsb = rows[:, CA:].reshape(NUM_WORKERS, RPW, CB)
    coff = ((tokens & 1) * D).reshape(NUM_WORKERS, RPW, L)
    coff = coff.transpose(0, 2, 1)
    lens = token_lengths.reshape(NUM_WORKERS, RPW)
    f = pl.kernel(
        _body,
        out_type=jax.ShapeDtypeStruct((B, D), jnp.float32),
        mesh=mesh,
        compiler_params=pltpu.CompilerParams(needs_layout_passes=False,
                                             use_tc_tiling_on_sc=True),
        scratch_types=[
            pltpu.VMEM((RPW, CA), jnp.int32),
            pltpu.VMEM((RPW, CB), jnp.int32),
            pltpu.VMEM((L, RPW), jnp.int32),
            pltpu.VMEM((RPW,), jnp.int32),
            pltpu.VMEM((RPW,), jnp.float32),
            pltpu.VMEM((2, CA, 2 * D), jnp.float32),
            pltpu.VMEM((2, CB, 2 * D), jnp.float32),
            pltpu.VMEM((RPW, D), jnp.float32),
            (pltpu.SemaphoreType.DMA, pltpu.SemaphoreType.DMA),
        ],
        interpret=interpret,
    )
    return f(rowsa, rowsb, coff, lens, tab2)


def kernel(tokens, token_lengths, table):
    return _run(tokens, token_lengths, table)
